# Initial kernel scaffold; baseline (speedup 1.0000x reference)
import jax
import jax.numpy as jnp
from jax.experimental import pallas as pl


def kernel(x_sparse, x_varlen, x_dense, tables):
    raise NotImplementedError("write your pallas kernel here")


# speedup vs baseline: 1.0235x; 1.0235x over previous
import functools

import jax
import jax.numpy as jnp
from jax import lax
from jax.experimental import pallas as pl
from jax.experimental.pallas import tpu as pltpu
from jax.experimental.pallas import tpu_sc as plsc

B = 16384
F = 26
V = 100000
D = 64
N = B * F


def _build_sc_gather():
    info = plsc.get_sparse_core_info()
    NC, NS, L = info.num_cores, info.num_subcores, info.num_lanes
    NW = NC * NS
    n_per_w = N // NW
    CH = 128
    n_chunks = n_per_w // CH
    mesh = plsc.VectorSubcoreMesh(core_axis_name="c", subcore_axis_name="s")

    @functools.partial(
        pl.kernel,
        mesh=mesh,
        compiler_params=pltpu.CompilerParams(use_tc_tiling_on_sc=False),
        out_type=jax.ShapeDtypeStruct((N, D), jnp.float32),
        scratch_types=[
            pltpu.VMEM((n_per_w,), jnp.int32),
            pltpu.VMEM((CH, D), jnp.float32),
            pltpu.VMEM((CH, D), jnp.float32),
            pltpu.SemaphoreType.DMA,
            pltpu.SemaphoreType.DMA,
        ],
    )
    def sc_gather(idx_hbm, tab_hbm, out_hbm, idx_v, buf0, buf1, sem0, sem1):
        wid = lax.axis_index("s") * NC + lax.axis_index("c")
        base = wid * n_per_w
        pltpu.sync_copy(idx_hbm.at[pl.ds(base, n_per_w)], idx_v)

        lane = lax.broadcasted_iota(jnp.int32, (L,), 0)

        def add_off(i, carry):
            pos = i * L + lane
            sl = pl.ds(i * L, L)
            idx_v[sl] = idx_v[sl] + lax.rem(pos, F) * V
            return carry

        lax.fori_loop(0, n_per_w // L, add_off, 0)

        bufs = (buf0, buf1)
        sems = (sem0, sem1)

        def gather(g, b):
            src = tab_hbm.at[idx_v.at[pl.ds(g * CH, CH)]]
            return pltpu.make_async_copy(src, bufs[b], sems[b])

        gather(0, 0).start()
        gather(1, 1).start()

        def body(g2, carry):
            for b in range(2):
                g = g2 * 2 + b
                gather(g, b).wait()
                pltpu.sync_copy(bufs[b], out_hbm.at[pl.ds(base + g * CH, CH)])
                gather(g + 2, b).start()
            return carry

        lax.fori_loop(0, n_chunks // 2 - 1, body, 0)

        for b in range(2):
            g = n_chunks - 2 + b
            gather(g, b).wait()
            pltpu.sync_copy(bufs[b], out_hbm.at[pl.ds(base + g * CH, CH)])

    return sc_gather


_sc_gather = _build_sc_gather()


def kernel(x_sparse, x_varlen, x_dense, tables):
    idx_flat = x_sparse.reshape(N)
    tab_flat = tables.reshape(F * V, D)
    out = _sc_gather(idx_flat, tab_flat)
    return out.reshape(B, F, D)


# speedup vs baseline: 2.4783x; 2.4215x over previous
import functools

import jax
import jax.numpy as jnp
from jax import lax
from jax.experimental import pallas as pl
from jax.experimental.pallas import tpu as pltpu
from jax.experimental.pallas import tpu_sc as plsc

B = 16384
F = 26
V = 100000
D = 64
R = F * D


def _build_sc_gather():
    info = plsc.get_sparse_core_info()
    NC, NS, L = info.num_cores, info.num_subcores, info.num_lanes
    NW = NC * NS
    rows_per_w = R // NW
    HB = B // 2
    G = HB // L
    mesh = plsc.VectorSubcoreMesh(core_axis_name="c", subcore_axis_name="s")

    @functools.partial(
        pl.kernel,
        mesh=mesh,
        compiler_params=pltpu.CompilerParams(
            use_tc_tiling_on_sc=True, needs_layout_passes=False),
        out_type=jax.ShapeDtypeStruct((R, B), jnp.float32),
        scratch_types=[
            pltpu.VMEM((V,), jnp.float32),
            pltpu.VMEM((B,), jnp.int32),
            pltpu.VMEM((HB,), jnp.float32),
        ],
    )
    def sc_gather(xsT_hbm, tab_hbm, out_hbm, slab_v, idx_v, out_v):
        wid = lax.axis_index("s") * NC + lax.axis_index("c")
        row0 = wid * rows_per_w

        def row_body(r, carry):
            f = r // D

            @pl.when(jnp.logical_or(r == row0, r % D == 0))
            def _load_idx():
                pltpu.sync_copy(xsT_hbm.at[f], idx_v)

            pltpu.sync_copy(tab_hbm.at[r], slab_v)
            for half in range(2):
                base = half * HB

                def g_body(i, c):
                    out_v[pl.ds(i * L, L)] = plsc.load_gather(
                        slab_v, [idx_v[pl.ds(base + i * L, L)]])
                    return c

                lax.fori_loop(0, G, g_body, 0, unroll=8)
                pltpu.sync_copy(out_v, out_hbm.at[r, pl.ds(base, HB)])
            return carry

        lax.fori_loop(row0, row0 + rows_per_w, row_body, 0)

    return sc_gather


_sc_gather = _build_sc_gather()


def kernel(x_sparse, x_varlen, x_dense, tables):
    xsT = x_sparse.T
    tabT = tables.transpose(0, 2, 1).reshape(R, V)
    out_T = _sc_gather(xsT, tabT)
    return out_T.reshape(F, D, B).transpose(2, 0, 1)


# speedup vs baseline: 5.5524x; 2.2404x over previous
import functools

import jax
import jax.numpy as jnp
from jax import lax
from jax.experimental import pallas as pl
from jax.experimental.pallas import tpu as pltpu
from jax.experimental.pallas import tpu_sc as plsc

B = 16384
F = 26
V = 100000
D = 64
R = F * D


def _build_sc_gather():
    info = plsc.get_sparse_core_info()
    NC, NS, L = info.num_cores, info.num_subcores, info.num_lanes
    NW = NC * NS
    rows_per_w = R // NW
    QB = B // 4
    G = QB // L
    mesh = plsc.VectorSubcoreMesh(core_axis_name="c", subcore_axis_name="s")

    @functools.partial(
        pl.kernel,
        mesh=mesh,
        compiler_params=pltpu.CompilerParams(
            use_tc_tiling_on_sc=True, needs_layout_passes=False),
        out_type=jax.ShapeDtypeStruct((R, B), jnp.float32),
        scratch_types=[
            pltpu.VMEM((V,), jnp.float32),
            pltpu.VMEM((B,), jnp.int32),
            pltpu.VMEM((QB,), jnp.float32),
            pltpu.VMEM((QB,), jnp.float32),
            pltpu.SemaphoreType.DMA,
            pltpu.SemaphoreType.DMA,
        ],
    )
    def sc_gather(xsT_hbm, tab_hbm, out_hbm, slab_v, idx_v, ob0, ob1, sm0, sm1):
        wid = lax.axis_index("s") * NC + lax.axis_index("c")
        row0 = wid * rows_per_w
        obufs = (ob0, ob1)
        sems = (sm0, sm1)

        def out_copy(r, q, p):
            return pltpu.make_async_copy(
                obufs[p], out_hbm.at[r, pl.ds(q * QB, QB)], sems[p])

        def row_body(r, carry):
            f = r // D

            @pl.when(jnp.logical_or(r == row0, r % D == 0))
            def _load_idx():
                pltpu.sync_copy(xsT_hbm.at[f], idx_v)

            pltpu.sync_copy(tab_hbm.at[r], slab_v)
            for q in range(4):
                p = q % 2
                if q >= 2:
                    out_copy(r, q - 2, p).wait()
                else:

                    @pl.when(r > row0)
                    def _drain():
                        out_copy(r - 1, q + 2, p).wait()

                base = q * QB
                ob = obufs[p]

                @plsc.parallel_loop(0, G, unroll=8)
                def _gather(i):
                    ob[pl.ds(i * L, L)] = plsc.load_gather(
                        slab_v, [idx_v[pl.ds(base + i * L, L)]])

                out_copy(r, q, p).start()
            return carry

        lax.fori_loop(row0, row0 + rows_per_w, row_body, 0)
        rlast = row0 + rows_per_w - 1
        for q in range(2, 4):
            out_copy(rlast, q, q % 2).wait()

    return sc_gather


_sc_gather = _build_sc_gather()


def kernel(x_sparse, x_varlen, x_dense, tables):
    xsT = x_sparse.T
    tabT = tables.transpose(0, 2, 1).reshape(R, V)
    out_T = _sc_gather(xsT, tabT)
    return out_T.reshape(F, D, B).transpose(2, 0, 1)


# speedup vs baseline: 5.5549x; 1.0005x over previous
import functools

import jax
import jax.numpy as jnp
from jax import lax
from jax.experimental import pallas as pl
from jax.experimental.pallas import tpu as pltpu
from jax.experimental.pallas import tpu_sc as plsc

B = 16384
F = 26
V = 100000
D = 64
R = F * D


def _build_sc_gather():
    info = plsc.get_sparse_core_info()
    NC, NS, L = info.num_cores, info.num_subcores, info.num_lanes
    NW = NC * NS
    rows_per_w = R // NW
    QB = B // 4
    G = QB // L
    CSZ = (33280, 33280, 33408)
    COF = (0, 33280, 66560)
    TOF = 99968
    mesh = plsc.VectorSubcoreMesh(core_axis_name="c", subcore_axis_name="s")

    @functools.partial(
        pl.kernel,
        mesh=mesh,
        compiler_params=pltpu.CompilerParams(
            use_tc_tiling_on_sc=True, needs_layout_passes=False),
        out_type=jax.ShapeDtypeStruct((R, B), jnp.float32),
        scratch_types=[
            pltpu.VMEM((V,), jnp.float32),
            pltpu.VMEM((B,), jnp.int32),
            pltpu.VMEM((QB,), jnp.float32),
            pltpu.VMEM((QB,), jnp.float32),
            pltpu.VMEM((2 * L,), jnp.float32),
            pltpu.SemaphoreType.DMA,
            pltpu.SemaphoreType.DMA,
            pltpu.SemaphoreType.DMA,
        ],
    )
    def sc_gather(xsT_hbm, tab_hbm, out_hbm, slab_v, idx_v, ob0, ob1, tail_v,
                  sm0, sm1, smr):
        wid = lax.axis_index("s") * NC + lax.axis_index("c")
        row0 = wid * rows_per_w
        obufs = (ob0, ob1)
        sems = (sm0, sm1)

        def out_copy(r, q, p):
            return pltpu.make_async_copy(
                obufs[p], out_hbm.at[r, pl.ds(q * QB, QB)], sems[p])

        def row_copies(r):
            cps = [pltpu.make_async_copy(
                tab_hbm.at[r, pl.ds(COF[c], CSZ[c])],
                slab_v.at[pl.ds(COF[c], CSZ[c])], smr) for c in range(3)]
            cps.append(pltpu.make_async_copy(
                tab_hbm.at[r, pl.ds(TOF, 2 * L)], tail_v, smr))
            return cps

        def row_body(r, carry):
            f = r // D

            @pl.when(jnp.logical_or(r == row0, r % D == 0))
            def _load_idx():
                pltpu.sync_copy(xsT_hbm.at[f], idx_v)

            for cp in row_copies(r):
                cp.start()
            for cp in row_copies(r):
                cp.wait()
            for t in range(2):
                slab_v[pl.ds(TOF + t * L, L)] = tail_v[pl.ds(t * L, L)]
            for q in range(4):
                p = q % 2
                if q >= 2:
                    out_copy(r, q - 2, p).wait()
                else:

                    @pl.when(r > row0)
                    def _drain():
                        out_copy(r - 1, q + 2, p).wait()

                base = q * QB
                ob = obufs[p]

                @plsc.parallel_loop(0, G, unroll=8)
                def _gather(i):
                    ob[pl.ds(i * L, L)] = plsc.load_gather(
                        slab_v, [idx_v[pl.ds(base + i * L, L)]])

                out_copy(r, q, p).start()
            return carry

        lax.fori_loop(row0, row0 + rows_per_w, row_body, 0)
        rlast = row0 + rows_per_w - 1
        for q in range(2, 4):
            out_copy(rlast, q, q % 2).wait()

    return sc_gather


_sc_gather = _build_sc_gather()


def kernel(x_sparse, x_varlen, x_dense, tables):
    xsT = x_sparse.T
    tabT = tables.transpose(0, 2, 1).reshape(R, V)
    out_T = _sc_gather(xsT, tabT)
    return out_T.reshape(F, D, B).transpose(2, 0, 1)
